# Initial kernel scaffold; baseline (speedup 1.0000x reference)
import jax
import jax.numpy as jnp
from jax.experimental import pallas as pl


def kernel(x, Wr, br, W1, b1, W2, b2):
    raise NotImplementedError("write your pallas kernel here")


# speedup vs baseline: 3.0718x; 3.0718x over previous
import jax
import jax.numpy as jnp
from jax.experimental import pallas as pl
from jax.experimental.pallas import tpu as pltpu

B, S, D, E, F, K = 1, 2048, 768, 8, 1536, 2
N = S * K
T = 128
MAX_TILES = N // T + E
NSLOT = MAX_TILES * T


def _ffn_body(te_ref, na_ref, xs_ref, w1_ref, b1_ref, w2_ref, b2_ref, y_ref):
    i = pl.program_id(0)

    @pl.when(i < na_ref[0])
    def _():
        h = jnp.dot(xs_ref[...], w1_ref[0], preferred_element_type=jnp.float32)
        h = h + b1_ref[0, 0]
        h = h * 0.5 * (1.0 + jax.lax.erf(h * 0.7071067811865476))
        y = jnp.dot(h.astype(jnp.bfloat16), w2_ref[0],
                    preferred_element_type=jnp.float32)
        y_ref[...] = y + b2_ref[0, 0]


def _clamp(i, na_ref):
    return jnp.minimum(i, na_ref[0] - 1)


def _grouped_ffn(xs, w1, b1, w2, b2, te, na):
    grid_spec = pltpu.PrefetchScalarGridSpec(
        num_scalar_prefetch=2,
        grid=(MAX_TILES,),
        in_specs=[
            pl.BlockSpec((T, D), lambda i, te, na: (_clamp(i, na), 0)),
            pl.BlockSpec((1, D, F), lambda i, te, na: (te[_clamp(i, na)], 0, 0)),
            pl.BlockSpec((1, 1, F), lambda i, te, na: (te[_clamp(i, na)], 0, 0)),
            pl.BlockSpec((1, F, D), lambda i, te, na: (te[_clamp(i, na)], 0, 0)),
            pl.BlockSpec((1, 1, D), lambda i, te, na: (te[_clamp(i, na)], 0, 0)),
        ],
        out_specs=pl.BlockSpec((T, D), lambda i, te, na: (_clamp(i, na), 0)),
    )
    return pl.pallas_call(
        _ffn_body,
        grid_spec=grid_spec,
        out_shape=jax.ShapeDtypeStruct((NSLOT, D), jnp.float32),
    )(te, na, xs, w1, b1, w2, b2)


def kernel(x, Wr, br, W1, b1, W2, b2):
    x2 = x.reshape(S, D)

    logits = jnp.einsum('bsd,de->bse', x, Wr) + br
    probs = jax.nn.softmax(logits, axis=-1)
    tkp, tki = jax.lax.top_k(probs, K)
    gates = tkp / jnp.sum(tkp, axis=-1, keepdims=True)

    e_flat = tki.reshape(N)
    onehot = (e_flat[:, None] == jnp.arange(E)[None, :]).astype(jnp.int32)
    csum = jnp.cumsum(onehot, axis=0)
    counts = csum[-1]
    rank = jnp.take_along_axis(csum - onehot, e_flat[:, None], axis=1)[:, 0]
    padded = ((counts + T - 1) // T) * T
    ends = jnp.cumsum(padded)
    offs = ends - padded
    pos = offs[e_flat] + rank
    tok = jnp.arange(N, dtype=jnp.int32) // K
    row_ids = jnp.zeros((NSLOT,), jnp.int32).at[pos].set(tok)
    na = (ends[-1:] // T).astype(jnp.int32)
    tile_start = jnp.arange(MAX_TILES, dtype=jnp.int32) * T
    te = jnp.minimum((tile_start[:, None] >= ends[None, :]).sum(axis=1),
                     E - 1).astype(jnp.int32)

    xs = jnp.take(x2.astype(jnp.bfloat16), row_ids, axis=0)
    y = _grouped_ffn(xs, W1.astype(jnp.bfloat16), b1.reshape(E, 1, F),
                     W2.astype(jnp.bfloat16), b2.reshape(E, 1, D), te, na)
    posr = pos.reshape(S, K)
    g = gates.reshape(S, K)
    out = (jnp.take(y, posr[:, 0], axis=0) * g[:, :1]
           + jnp.take(y, posr[:, 1], axis=0) * g[:, 1:])
    return out.reshape(B, S, D)


# speedup vs baseline: 3.2002x; 1.0418x over previous
import jax
import jax.numpy as jnp
from jax.experimental import pallas as pl
from jax.experimental.pallas import tpu as pltpu

B, S, D, E, F, K = 1, 2048, 768, 8, 1536, 2
N = S * K
T = 128
MAX_TILES = N // T + E
NSLOT = MAX_TILES * T


def _ffn_body(te_ref, na_ref, xs_ref, w1_ref, b1_ref, w2_ref, b2_ref, y_ref):
    i = pl.program_id(0)

    @pl.when(i < na_ref[0])
    def _():
        h = jnp.dot(xs_ref[...], w1_ref[0], preferred_element_type=jnp.float32)
        h = h + b1_ref[0, 0]
        h = h * 0.5 * (1.0 + jax.lax.erf(h * 0.7071067811865476))
        y = jnp.dot(h.astype(jnp.bfloat16), w2_ref[0],
                    preferred_element_type=jnp.float32)
        y_ref[...] = y + b2_ref[0, 0]


def _clamp(i, na_ref):
    return jnp.minimum(i, na_ref[0] - 1)


def _grouped_ffn(xs, w1, b1, w2, b2, te, na):
    grid_spec = pltpu.PrefetchScalarGridSpec(
        num_scalar_prefetch=2,
        grid=(MAX_TILES,),
        in_specs=[
            pl.BlockSpec((T, D), lambda i, te, na: (_clamp(i, na), 0)),
            pl.BlockSpec((1, D, F), lambda i, te, na: (te[_clamp(i, na)], 0, 0)),
            pl.BlockSpec((1, 1, F), lambda i, te, na: (te[_clamp(i, na)], 0, 0)),
            pl.BlockSpec((1, F, D), lambda i, te, na: (te[_clamp(i, na)], 0, 0)),
            pl.BlockSpec((1, 1, D), lambda i, te, na: (te[_clamp(i, na)], 0, 0)),
        ],
        out_specs=pl.BlockSpec((T, D), lambda i, te, na: (_clamp(i, na), 0)),
    )
    return pl.pallas_call(
        _ffn_body,
        grid_spec=grid_spec,
        out_shape=jax.ShapeDtypeStruct((NSLOT, D), jnp.float32),
    )(te, na, xs, w1, b1, w2, b2)


def kernel(x, Wr, br, W1, b1, W2, b2):
    x2 = x.reshape(S, D)

    logits = jnp.einsum('bsd,de->bse', x, Wr) + br
    probs = jax.nn.softmax(logits, axis=-1)
    tkp, tki = jax.lax.top_k(probs, K)
    gates = tkp / jnp.sum(tkp, axis=-1, keepdims=True)

    e_flat = tki.reshape(N)
    onehot = (e_flat[:, None] == jnp.arange(E)[None, :]).astype(jnp.int32)
    csum = jnp.cumsum(onehot, axis=0)
    counts = csum[-1]
    rank = jnp.take_along_axis(csum - onehot, e_flat[:, None], axis=1)[:, 0]
    padded = ((counts + T - 1) // T) * T
    ends = jnp.cumsum(padded)
    offs = ends - padded
    pos = offs[e_flat] + rank
    tok = jnp.arange(N, dtype=jnp.int32) // K
    row_ids = jnp.zeros((NSLOT,), jnp.int32).at[pos].add(tok)
    na = (ends[-1:] // T).astype(jnp.int32)
    tile_start = jnp.arange(MAX_TILES, dtype=jnp.int32) * T
    te = jnp.minimum((tile_start[:, None] >= ends[None, :]).sum(axis=1),
                     E - 1).astype(jnp.int32)

    xs = jnp.take(x2.astype(jnp.bfloat16), row_ids, axis=0)
    y = _grouped_ffn(xs, W1.astype(jnp.bfloat16), b1.reshape(E, 1, F),
                     W2.astype(jnp.bfloat16), b2.reshape(E, 1, D), te, na)
    posr = pos.reshape(S, K)
    g = gates.reshape(S, K)
    out = (jnp.take(y, posr[:, 0], axis=0) * g[:, :1]
           + jnp.take(y, posr[:, 1], axis=0) * g[:, 1:])
    return out.reshape(B, S, D)
